# baseline (speedup 1.0000x reference)
import functools

import jax
import jax.numpy as jnp
from jax import lax
from jax.experimental import pallas as pl
from jax.experimental.pallas import tpu as pltpu
from jax.experimental.pallas import tpu_sc as plsc

K = 100000
NQ = 16
D = 64
KB = 2000
NBT = 21
NW = 32
SC_BASE = 40096
PER_W = (K - SC_BASE) // NW
NCH = 3
CHR = PER_W // NCH
BIGF = 3.0e38


def _tc_dist_body(mem_ref, q_ref, bv_ref, bi_ref, bestv_scr, bidx_scr):
    i = pl.program_id(0)
    mem = mem_ref[...]
    q = q_ref[...]
    ones = jnp.ones((1, D), dtype=jnp.float32)
    norms = jax.lax.dot_general(
        ones, mem * mem, (((1,), (1,)), ((), ())),
        preferred_element_type=jnp.float32,
        precision=jax.lax.Precision.HIGHEST)
    dots = jax.lax.dot_general(
        q, mem, (((1,), (1,)), ((), ())),
        preferred_element_type=jnp.float32,
        precision=jax.lax.Precision.HIGHEST)
    dist = norms - 2.0 * dots
    bmin = jnp.min(dist, axis=1, keepdims=True)
    cols = jax.lax.broadcasted_iota(jnp.int32, (NQ, KB), 1) + i * KB
    bidx = jnp.min(jnp.where(dist == bmin, cols, K),
                   axis=1, keepdims=True)

    @pl.when(i == 0)
    def _init():
        bestv_scr[...] = bmin
        bidx_scr[...] = bidx

    @pl.when(i > 0)
    def _update():
        prev = bestv_scr[...]
        upd = bmin < prev
        bestv_scr[...] = jnp.where(upd, bmin, prev)
        bidx_scr[...] = jnp.where(upd, bidx, bidx_scr[...])

    @pl.when(i == NBT - 1)
    def _final():
        bv_ref[...] = bestv_scr[...]
        bi_ref[...] = bidx_scr[...]


def _tc_dist(in_memory, inpt):
    return pl.pallas_call(
        _tc_dist_body,
        grid=(NBT,),
        in_specs=[
            pl.BlockSpec((KB, D), lambda i: (i, 0)),
            pl.BlockSpec((NQ, D), lambda i: (0, 0)),
        ],
        out_specs=[
            pl.BlockSpec((NQ, 1), lambda i: (0, 0)),
            pl.BlockSpec((NQ, 1), lambda i: (0, 0)),
        ],
        out_shape=[
            jax.ShapeDtypeStruct((NQ, 1), jnp.float32),
            jax.ShapeDtypeStruct((NQ, 1), jnp.int32),
        ],
        scratch_shapes=[
            pltpu.VMEM((NQ, 1), jnp.float32),
            pltpu.VMEM((NQ, 1), jnp.int32),
        ],
        compiler_params=pltpu.CompilerParams(
            dimension_semantics=("arbitrary",)),
    )(in_memory, inpt)


def _rot(v, s):
    idx = (jnp.arange(16, dtype=jnp.int32) + s) % 16
    dnums = lax.GatherDimensionNumbers(
        offset_dims=(), collapsed_slice_dims=(0,), start_index_map=(0,))
    return lax.gather(v, idx[:, None], dnums, (1,),
                      mode=lax.GatherScatterMode.PROMISE_IN_BOUNDS)


def _allsum(v):
    for s in (8, 4, 2, 1):
        v = v + _rot(v, s)
    return v


@functools.cache
def _make_sc_dist():
    mesh = plsc.VectorSubcoreMesh(core_axis_name="c", subcore_axis_name="s")

    @functools.partial(
        pl.kernel,
        mesh=mesh,
        out_type=[
            jax.ShapeDtypeStruct((NW, NQ), jnp.float32),
            jax.ShapeDtypeStruct((NW, NQ), jnp.int32),
        ],
        scratch_types=[
            pltpu.VMEM((NQ, D), jnp.float32),
            pltpu.VMEM((CHR, D), jnp.float32),
            pltpu.VMEM((CHR, D), jnp.float32),
            pltpu.VMEM((NQ,), jnp.float32),
            pltpu.VMEM((NQ,), jnp.int32),
            pltpu.SemaphoreType.DMA,
            pltpu.SemaphoreType.DMA,
        ],
        compiler_params=pltpu.CompilerParams(use_tc_tiling_on_sc=False),
        cost_estimate=pl.CostEstimate(
            flops=4 * (K - SC_BASE) * NQ * D,
            bytes_accessed=(K - SC_BASE) * D * 4,
            transcendentals=0),
    )
    def _sc_dist(q_hbm, table_hbm, bv_hbm, bi_hbm,
                 q_v, buf0, buf1, resv_v, resi_v, sem0, sem1):
        wid = lax.axis_index("s") * 2 + lax.axis_index("c")
        base = SC_BASE + wid * PER_W
        pltpu.sync_copy(q_hbm, q_v)
        bufs = (buf0, buf1)
        sems = (sem0, sem1)
        lanes = jnp.arange(16, dtype=jnp.int32)

        copies = [None] * NCH
        copies[0] = pltpu.async_copy(
            table_hbm.at[pl.ds(base, CHR)], buf0, sem0)

        best_v = jnp.full((NQ,), BIGF, jnp.float32)
        best_i = jnp.full((NQ,), K, jnp.int32)

        for c in range(NCH):
            if c + 1 < NCH:
                copies[c + 1] = pltpu.async_copy(
                    table_hbm.at[pl.ds(base + (c + 1) * CHR, CHR)],
                    bufs[(c + 1) % 2], sems[(c + 1) % 2])
            copies[c].wait()
            buf = bufs[c % 2]
            cbase = base + c * CHR

            for half in range(2):
                q0 = half * 8
                qv = [[q_v[qi, pl.ds(16 * v, 16)] for v in range(4)]
                      for qi in range(q0, q0 + 8)]

                def body(rr, carry, q0=q0, qv=qv, buf=buf, cbase=cbase):
                    bv, bi = carry
                    for u in range(2):
                        r = rr * 2 + u
                        k0 = buf[r, pl.ds(0, 16)]
                        k1 = buf[r, pl.ds(16, 16)]
                        k2 = buf[r, pl.ds(32, 16)]
                        k3 = buf[r, pl.ds(48, 16)]
                        knv = _allsum(k0 * k0 + k1 * k1 + k2 * k2 + k3 * k3)
                        kidx = jnp.full((NQ,), cbase + r, jnp.int32)
                        for j in range(8):
                            w0, w1, w2, w3 = qv[j]
                            pv = k0 * w0 + k1 * w1 + k2 * w2 + k3 * w3
                            dv = knv - 2.0 * _allsum(pv)
                            m = (lanes == (q0 + j)) & (dv < bv)
                            bv = jnp.where(m, dv, bv)
                            bi = jnp.where(m, kidx, bi)
                    return bv, bi

                best_v, best_i = lax.fori_loop(
                    0, CHR // 2, body, (best_v, best_i))

        resv_v[...] = best_v
        resi_v[...] = best_i
        pltpu.sync_copy(resv_v, bv_hbm.at[wid])
        pltpu.sync_copy(resi_v, bi_hbm.at[wid])

    return _sc_dist


def _merge_body(tv_ref, ti_ref, sv_ref, si_ref, q_ref, bidx_ref, minv_ref):
    ident = (jax.lax.broadcasted_iota(jnp.int32, (NQ, NQ), 0) ==
             jax.lax.broadcasted_iota(jnp.int32, (NQ, NQ), 1)
             ).astype(jnp.float32)

    def _t(col):
        return jax.lax.dot_general(
            col, ident, (((0,), (0,)), ((), ())),
            preferred_element_type=jnp.float32,
            precision=jax.lax.Precision.HIGHEST).reshape(1, NQ)

    tv = _t(tv_ref[...])
    ti = _t(ti_ref[...].astype(jnp.float32))
    sv = sv_ref[...]
    si = si_ref[...]
    scv = jnp.min(sv, axis=0, keepdims=True)
    sci = jnp.min(jnp.where(sv == scv, si, K), axis=0, keepdims=True)
    take = (scv < tv) | ((scv == tv) & (sci.astype(jnp.float32) < ti))
    bestv = jnp.where(take, scv, tv)
    bidx = jnp.where(take, sci, ti.astype(jnp.int32))
    bidx_ref[...] = bidx
    q = q_ref[...]
    qnt = jax.lax.dot_general(
        jnp.ones((1, D), jnp.float32), q * q, (((1,), (1,)), ((), ())),
        preferred_element_type=jnp.float32,
        precision=jax.lax.Precision.HIGHEST)
    minv_ref[...] = jnp.min(bestv + qnt).reshape(1, 1)


def _merge(tv, ti, sv, si, inpt):
    return pl.pallas_call(
        _merge_body,
        out_shape=[
            jax.ShapeDtypeStruct((1, NQ), jnp.int32),
            jax.ShapeDtypeStruct((1, 1), jnp.float32),
        ],
    )(tv, ti, sv, si, inpt)


@functools.cache
def _make_sc_gather():
    mesh = plsc.VectorSubcoreMesh(core_axis_name="c", subcore_axis_name="s")

    @functools.partial(
        pl.kernel,
        mesh=mesh,
        out_type=jax.ShapeDtypeStruct((NQ, D), jnp.float32),
        scratch_types=[
            pltpu.VMEM((NQ,), jnp.int32),
            pltpu.VMEM((NQ, D), jnp.float32),
            pltpu.SemaphoreType.DMA,
        ],
        compiler_params=pltpu.CompilerParams(use_tc_tiling_on_sc=False),
    )
    def _sc_gather(idx_hbm, table_hbm, out_hbm, idx_v, rows_v, sem):
        wid = lax.axis_index("s") * 2 + lax.axis_index("c")

        @pl.when(wid == 0)
        def _():
            pltpu.sync_copy(idx_hbm, idx_v)
            pltpu.async_copy(table_hbm.at[idx_v], rows_v, sem).wait()
            pltpu.sync_copy(rows_v, out_hbm)

    return _sc_gather


def kernel(inpt, in_memory):
    sv, si = _make_sc_dist()(inpt, in_memory)
    tv, ti = _tc_dist(in_memory, inpt)
    bidx, minv = _merge(tv, ti, sv, si, inpt)
    matched = _make_sc_gather()(bidx.reshape(NQ), in_memory)
    return matched, minv[0, 0]


# speedup vs baseline: 1.2149x; 1.2149x over previous
import functools

import jax
import jax.numpy as jnp
from jax import lax
from jax.experimental import pallas as pl
from jax.experimental.pallas import tpu as pltpu
from jax.experimental.pallas import tpu_sc as plsc

K = 100000
NQ = 16
D = 64
KB = 2000
NBT = K // KB
BIGF = 3.0e38


def _tc_dist_body(mem_ref, q_ref, bi_ref, minv_ref, bestv_scr, bidx_scr):
    i = pl.program_id(0)
    mem = mem_ref[...]
    q = q_ref[...]
    ones = jnp.ones((1, D), dtype=jnp.float32)
    norms = jax.lax.dot_general(
        ones, mem * mem, (((1,), (1,)), ((), ())),
        preferred_element_type=jnp.float32,
        precision=jax.lax.Precision.HIGHEST)
    dots = jax.lax.dot_general(
        q, mem, (((1,), (1,)), ((), ())),
        preferred_element_type=jnp.float32,
        precision=jax.lax.Precision.HIGHEST)
    dist = norms - 2.0 * dots
    bmin = jnp.min(dist, axis=1, keepdims=True)
    cols = jax.lax.broadcasted_iota(jnp.int32, (NQ, KB), 1) + i * KB
    bidx = jnp.min(jnp.where(dist == bmin, cols, K),
                   axis=1, keepdims=True)

    @pl.when(i == 0)
    def _init():
        bestv_scr[...] = bmin
        bidx_scr[...] = bidx

    @pl.when(i > 0)
    def _update():
        prev = bestv_scr[...]
        upd = bmin < prev
        bestv_scr[...] = jnp.where(upd, bmin, prev)
        bidx_scr[...] = jnp.where(upd, bidx, bidx_scr[...])

    @pl.when(i == NBT - 1)
    def _final():
        ident = (jax.lax.broadcasted_iota(jnp.int32, (NQ, NQ), 0) ==
                 jax.lax.broadcasted_iota(jnp.int32, (NQ, NQ), 1)
                 ).astype(jnp.float32)

        def _t(col):
            return jax.lax.dot_general(
                col, ident, (((0,), (0,)), ((), ())),
                preferred_element_type=jnp.float32,
                precision=jax.lax.Precision.HIGHEST).reshape(1, NQ)

        bi_ref[...] = _t(bidx_scr[...].astype(jnp.float32)).astype(jnp.int32)
        qnt = jax.lax.dot_general(
            ones, q * q, (((1,), (1,)), ((), ())),
            preferred_element_type=jnp.float32,
            precision=jax.lax.Precision.HIGHEST)
        minv_ref[...] = jnp.min(_t(bestv_scr[...]) + qnt).reshape(1, 1)


def _tc_dist(in_memory, inpt):
    return pl.pallas_call(
        _tc_dist_body,
        grid=(NBT,),
        in_specs=[
            pl.BlockSpec((KB, D), lambda i: (i, 0)),
            pl.BlockSpec((NQ, D), lambda i: (0, 0)),
        ],
        out_specs=[
            pl.BlockSpec((1, NQ), lambda i: (0, 0)),
            pl.BlockSpec((1, 1), lambda i: (0, 0)),
        ],
        out_shape=[
            jax.ShapeDtypeStruct((1, NQ), jnp.int32),
            jax.ShapeDtypeStruct((1, 1), jnp.float32),
        ],
        scratch_shapes=[
            pltpu.VMEM((NQ, 1), jnp.float32),
            pltpu.VMEM((NQ, 1), jnp.int32),
        ],
        compiler_params=pltpu.CompilerParams(
            dimension_semantics=("arbitrary",)),
    )(in_memory, inpt)


@functools.cache
def _make_sc_gather():
    mesh = plsc.VectorSubcoreMesh(core_axis_name="c", subcore_axis_name="s")

    @functools.partial(
        pl.kernel,
        mesh=mesh,
        out_type=jax.ShapeDtypeStruct((NQ, D), jnp.float32),
        scratch_types=[
            pltpu.VMEM((NQ,), jnp.int32),
            pltpu.VMEM((NQ, D), jnp.float32),
            pltpu.SemaphoreType.DMA,
        ],
        compiler_params=pltpu.CompilerParams(use_tc_tiling_on_sc=False),
    )
    def _sc_gather(idx_hbm, table_hbm, out_hbm, idx_v, rows_v, sem):
        wid = lax.axis_index("s") * 2 + lax.axis_index("c")

        @pl.when(wid == 0)
        def _():
            pltpu.sync_copy(idx_hbm, idx_v)
            pltpu.async_copy(table_hbm.at[idx_v], rows_v, sem).wait()
            pltpu.sync_copy(rows_v, out_hbm)

    return _sc_gather


def kernel(inpt, in_memory):
    bidx, minv = _tc_dist(in_memory, inpt)
    matched = _make_sc_gather()(bidx.reshape(NQ), in_memory)
    return matched, minv[0, 0]


# speedup vs baseline: 1.9433x; 1.5996x over previous
import jax
import jax.numpy as jnp
from jax.experimental import pallas as pl
from jax.experimental.pallas import tpu as pltpu

K = 100000
KB = 20000
NB = K // KB
NQ = 16
D = 64


def _nn_body(mem_ref, q_ref, matched_ref, minv_ref, bestv_scr):
    i = pl.program_id(0)
    mem = mem_ref[...]
    q = q_ref[...]
    ones = jnp.ones((1, D), dtype=jnp.float32)
    msq = mem * mem
    norms = jax.lax.dot_general(
        ones, msq, (((1,), (1,)), ((), ())),
        preferred_element_type=jnp.float32,
        precision=jax.lax.Precision.HIGHEST)
    dots = jax.lax.dot_general(
        q, mem, (((1,), (1,)), ((), ())),
        preferred_element_type=jnp.float32,
        precision=jax.lax.Precision.HIGHEST)
    dist = norms - 2.0 * dots
    bmin = jnp.min(dist, axis=1, keepdims=True)
    cols = jax.lax.broadcasted_iota(jnp.int32, (NQ, KB), 1)
    onehot = jnp.where(dist == bmin, jnp.float32(1.0), jnp.float32(0.0))
    bcol = jnp.min(jnp.where(dist == bmin, cols, K), axis=1, keepdims=True)
    onehot = jnp.where(cols == bcol, onehot, jnp.float32(0.0))
    rowsel = jax.lax.dot_general(
        onehot, mem, (((1,), (0,)), ((), ())),
        preferred_element_type=jnp.float32)

    @pl.when(i == 0)
    def _init():
        bestv_scr[...] = bmin
        matched_ref[...] = rowsel

    @pl.when(i > 0)
    def _update():
        prev = bestv_scr[...]
        upd = bmin < prev
        bestv_scr[...] = jnp.where(upd, bmin, prev)
        matched_ref[...] = jnp.where(
            jnp.broadcast_to(upd, (NQ, D)), rowsel, matched_ref[...])

    @pl.when(i == NB - 1)
    def _final():
        qn = jnp.sum(q * q, axis=1, keepdims=True)
        minv_ref[...] = jnp.min(bestv_scr[...] + qn).reshape(1, 1)


def kernel(inpt, in_memory):
    matched, minv = pl.pallas_call(
        _nn_body,
        grid=(NB,),
        in_specs=[
            pl.BlockSpec((KB, D), lambda i: (i, 0)),
            pl.BlockSpec((NQ, D), lambda i: (0, 0)),
        ],
        out_specs=[
            pl.BlockSpec((NQ, D), lambda i: (0, 0)),
            pl.BlockSpec((1, 1), lambda i: (0, 0)),
        ],
        out_shape=[
            jax.ShapeDtypeStruct((NQ, D), jnp.float32),
            jax.ShapeDtypeStruct((1, 1), jnp.float32),
        ],
        scratch_shapes=[pltpu.VMEM((NQ, 1), jnp.float32)],
        compiler_params=pltpu.CompilerParams(
            dimension_semantics=("arbitrary",)),
    )(in_memory, inpt)
    return matched, minv[0, 0]
